# baseline (speedup 1.0000x reference)
import dataclasses
import functools

import jax
import jax.numpy as jnp
from jax import lax
from jax.experimental import pallas as pl
from jax.experimental.pallas import tpu as pltpu
from jax.experimental.pallas import tpu_sc as plsc

EMBED_DIM = 1024
NUM_CORES = 2
NUM_SUBCORES = 16
NUM_WORKERS = NUM_CORES * NUM_SUBCORES
CHUNK = 32
SBUF = 2
GROUP = 16
SLOTS = 3
STREAM_ROWS = 288
L = 16


def kernel(positions, weights):
    b, s = positions.shape
    n = b * s
    flat_idx = positions.reshape(n).astype(jnp.int32)
    b_per_w = n // NUM_WORKERS
    dma_rows = b_per_w - STREAM_ROWS
    n_chunks = STREAM_ROWS // CHUNK
    n_groups = dma_rows // GROUP

    mesh = plsc.VectorSubcoreMesh(core_axis_name="c", subcore_axis_name="s")
    cp = pltpu.CompilerParams()
    if "needs_layout_passes" in pltpu.CompilerParams.__dataclass_fields__:
        cp = dataclasses.replace(cp, needs_layout_passes=False)

    @functools.partial(
        pl.kernel,
        mesh=mesh,
        compiler_params=cp,
        out_type=jax.ShapeDtypeStruct((n, EMBED_DIM), weights.dtype),
        scratch_types=[
            pltpu.VMEM((b_per_w,), jnp.int32),
            pltpu.VMEM((SBUF, CHUNK, EMBED_DIM), jnp.float32),
            pltpu.VMEM_SHARED((NUM_SUBCORES, SLOTS * GROUP, EMBED_DIM),
                              jnp.float32),
            pltpu.SemaphoreType.DMA((SBUF,)),
            pltpu.SemaphoreType.DMA((SBUF,)),
            pltpu.SemaphoreType.DMA((SLOTS,)),
            pltpu.SemaphoreType.DMA((SLOTS,)),
        ],
    )
    def gather_kernel(table_hbm, idx_hbm, out_hbm, idx_v, sbuf, stage,
                      gsem, wsem, dsem, fsem):
        cid = lax.axis_index("c")
        sid = lax.axis_index("s")
        wid = sid * NUM_CORES + cid
        base = wid * b_per_w
        dbase = base + STREAM_ROWS

        pltpu.sync_copy(idx_hbm.at[pl.ds(base, b_per_w)], idx_v)

        def sgather(cc, bi):
            return pltpu.make_async_copy(
                table_hbm.at[idx_v.at[pl.ds(cc * CHUNK, CHUNK)]],
                sbuf.at[bi],
                gsem.at[bi],
            )

        def swrite(cc, bi):
            return pltpu.make_async_copy(
                sbuf.at[bi],
                out_hbm.at[pl.ds(base + cc * CHUNK, CHUNK)],
                wsem.at[bi],
            )

        def fill(grp, slot):
            vec = idx_v[pl.ds(STREAM_ROWS + grp * GROUP, GROUP)]
            for i in range(GROUP):
                pltpu.make_async_copy(
                    table_hbm.at[pl.ds(vec[i], 1)],
                    stage.at[sid, pl.ds(slot * GROUP + i, 1)],
                    dsem.at[slot],
                ).start()

        def forward(grp, slot):
            return pltpu.make_async_copy(
                stage.at[sid, pl.ds(slot * GROUP, GROUP)],
                out_hbm.at[pl.ds(dbase + grp * GROUP, GROUP)],
                fsem.at[slot],
            )

        def drain_rows(slot):
            for _ in range(GROUP):
                pltpu.make_async_copy(
                    table_hbm.at[pl.ds(0, 1)],
                    stage.at[sid, pl.ds(slot * GROUP, 1)],
                    dsem.at[slot],
                ).wait()

        for bi in range(SBUF):
            sgather(bi, bi).start()

        for t in range(max(n_groups, n_chunks)):
            slot = t % SLOTS
            if t < n_groups:
                if t >= SLOTS:
                    forward(t - SLOTS, slot).wait()
                fill(t, slot)
            if t < n_chunks:
                bi = t % SBUF
                sgather(t, bi).wait()
                swrite(t, bi).start()
                if t + SBUF < n_chunks:
                    swrite(t, bi).wait()
                    sgather(t + SBUF, bi).start()
            if t < n_groups:
                drain_rows(slot)
                forward(t, slot).start()

        for k in range(min(SBUF, n_chunks)):
            swrite(0, (n_chunks - 1 - k) % SBUF).wait()
        for k in range(min(SLOTS, n_groups)):
            forward(0, (n_groups - 1 - k) % SLOTS).wait()

    out = gather_kernel(weights, flat_idx)
    return out.reshape(b, s, EMBED_DIM)


# speedup vs baseline: 1.1043x; 1.1043x over previous
import functools

import jax
import jax.numpy as jnp
from jax import lax
from jax.experimental import pallas as pl
from jax.experimental.pallas import tpu as pltpu
from jax.experimental.pallas import tpu_sc as plsc

EMBED_DIM = 1024
NUM_CORES = 2
NUM_SUBCORES = 16
NUM_WORKERS = NUM_CORES * NUM_SUBCORES
CHUNK = 16
NBUF = 4


def kernel(positions, weights):
    b, s = positions.shape
    n = b * s
    flat_idx = positions.reshape(n).astype(jnp.int32)
    b_per_w = n // NUM_WORKERS
    n_chunks = b_per_w // CHUNK

    mesh = plsc.VectorSubcoreMesh(core_axis_name="c", subcore_axis_name="s")

    @functools.partial(
        pl.kernel,
        mesh=mesh,
        out_type=jax.ShapeDtypeStruct((n, EMBED_DIM), weights.dtype),
        scratch_types=[
            pltpu.VMEM((b_per_w,), jnp.int32),
            pltpu.VMEM((NBUF, CHUNK, EMBED_DIM), jnp.float32),
            pltpu.SemaphoreType.DMA((NBUF,)),
            pltpu.SemaphoreType.DMA((NBUF,)),
        ],
    )
    def gather_kernel(table_hbm, idx_hbm, out_hbm, idx_v, rows_v, gsem, wsem):
        wid = lax.axis_index("s") * NUM_CORES + lax.axis_index("c")
        base = wid * b_per_w
        pltpu.sync_copy(idx_hbm.at[pl.ds(base, b_per_w)], idx_v)

        def gather(cc, bi):
            return pltpu.make_async_copy(
                table_hbm.at[idx_v.at[pl.ds(cc * CHUNK, CHUNK)]],
                rows_v.at[bi],
                gsem.at[bi],
            )

        def writeback(cc, bi):
            return pltpu.make_async_copy(
                rows_v.at[bi],
                out_hbm.at[pl.ds(base + cc * CHUNK, CHUNK)],
                wsem.at[bi],
            )

        for bi in range(NBUF):
            gather(bi, bi).start()

        @pl.loop(0, n_chunks, step=NBUF)
        def _(c):
            for bi in range(NBUF):
                cc = c + bi
                gather(cc, bi).wait()
                writeback(cc, bi).start()

                @pl.when(cc + NBUF < n_chunks)
                def _():
                    writeback(cc, bi).wait()
                    gather(cc + NBUF, bi).start()

        for bi in range(NBUF):
            writeback(n_chunks - NBUF + bi, bi).wait()

    out = gather_kernel(weights, flat_idx)
    return out.reshape(b, s, EMBED_DIM)
